# Initial kernel scaffold; baseline (speedup 1.0000x reference)
import jax
import jax.numpy as jnp
from jax.experimental import pallas as pl


def kernel(feat, edge_index, etypes, ntypes, weight, m_bias, loop_weight, h_bias):
    raise NotImplementedError("write your pallas kernel here")


# speedup vs baseline: 9.0824x; 9.0824x over previous
import functools

import jax
import jax.numpy as jnp
from jax import lax
from jax.experimental import pallas as pl
from jax.experimental.pallas import tpu as pltpu
from jax.experimental.pallas import tpu_sc as plsc

_NC = 2
_NS = 16
_NW = _NC * _NS
_CH = 128


def _project_kernel(f_ref, w_ref, b_ref, y_ref):
    y = jnp.dot(f_ref[...], w_ref[0],
                preferred_element_type=jnp.float32,
                precision=lax.Precision.HIGHEST)
    y_ref[...] = (y + b_ref[0])[None]


def _build_table(feat, w_all, b_all, bn):
    n, in_feat = feat.shape
    k, _, out_feat = w_all.shape
    nb = n // bn
    return pl.pallas_call(
        _project_kernel,
        grid=(nb, k),
        in_specs=[
            pl.BlockSpec((bn, in_feat), lambda i, j: (i, 0)),
            pl.BlockSpec((1, in_feat, out_feat), lambda i, j: (j, 0, 0)),
            pl.BlockSpec((1, 1, out_feat), lambda i, j: (j, 0, 0)),
        ],
        out_specs=pl.BlockSpec((1, bn, out_feat), lambda i, j: (j, i, 0)),
        out_shape=jax.ShapeDtypeStruct((k, n, out_feat), jnp.float32),
    )(feat, w_all, b_all)


def _combine_kernel(p_ref, o_ref):
    o_ref[...] = p_ref[0] + p_ref[1]


def _combine(partials, bn):
    _, n, out_feat = partials.shape
    return pl.pallas_call(
        _combine_kernel,
        grid=(n // bn,),
        in_specs=[pl.BlockSpec((2, bn, out_feat), lambda i: (0, i, 0))],
        out_specs=pl.BlockSpec((bn, out_feat), lambda i: (i, 0)),
        out_shape=jax.ShapeDtypeStruct((n, out_feat), jnp.float32),
    )(partials)


def _make_edge_kernel(out_feat, nch, acc_rows, rows_per_tile):
    mesh = plsc.VectorSubcoreMesh(core_axis_name="c", subcore_axis_name="s")

    @functools.partial(
        pl.kernel,
        mesh=mesh,
        out_type=jax.ShapeDtypeStruct((_NC, acc_rows, out_feat), jnp.float32),
        scratch_types=[
            pltpu.VMEM((nch, _CH), jnp.int32),
            pltpu.VMEM((nch, _CH), jnp.int32),
            pltpu.VMEM((_CH, out_feat), jnp.float32),
            pltpu.VMEM_SHARED((acc_rows, out_feat), jnp.float32),
            pltpu.SemaphoreType.DMA,
        ],
    )
    def edge_kernel(table_hbm, gidx_hbm, didx_hbm, zeros_hbm, out_hbm,
                    gidx_v, didx_v, rows_v, acc, sem):
        c = lax.axis_index("c")
        s = lax.axis_index("s")
        wid = s * _NC + c

        z0 = s * rows_per_tile
        pltpu.sync_copy(zeros_hbm.at[pl.ds(0, rows_per_tile)],
                        acc.at[pl.ds(z0, rows_per_tile)])
        pltpu.sync_copy(gidx_hbm.at[wid], gidx_v)
        pltpu.sync_copy(didx_hbm.at[wid], didx_v)
        plsc.subcore_barrier()

        def body(j, carry):
            pltpu.async_copy(table_hbm.at[gidx_v.at[j]], rows_v, sem).wait()
            pltpu.sync_copy(rows_v, acc.at[didx_v.at[j]], add=True)
            return carry

        lax.fori_loop(0, nch, body, 0)
        plsc.subcore_barrier()

        r0 = s * rows_per_tile
        pltpu.sync_copy(acc.at[pl.ds(r0, rows_per_tile)],
                        out_hbm.at[c, pl.ds(r0, rows_per_tile)])

    return edge_kernel


def kernel(feat, edge_index, etypes, ntypes, weight, m_bias, loop_weight, h_bias):
    n, in_feat = feat.shape
    num_rels, _, out_feat = weight.shape
    num_nt = loop_weight.shape[0]
    e = edge_index.shape[1]

    w_all = jnp.concatenate([weight, loop_weight], axis=0)
    b_all = jnp.concatenate([m_bias, h_bias[:, 0, :]], axis=0)[:, None, :]

    src = edge_index[0].astype(jnp.int32)
    dst = edge_index[1].astype(jnp.int32)
    node_ids = jnp.arange(n, dtype=jnp.int32)
    gidx = jnp.concatenate([
        etypes.astype(jnp.int32) * n + src,
        (num_rels + ntypes.astype(jnp.int32)) * n + node_ids,
    ])
    didx = jnp.concatenate([dst, node_ids])

    e_tot = e + n
    pw = -(-e_tot // (_NW * _CH)) * _CH
    nch = pw // _CH
    ep = _NW * pw
    gidx = jnp.concatenate([gidx, jnp.zeros((ep - e_tot,), jnp.int32)])
    didx = jnp.concatenate([didx, jnp.full((ep - e_tot,), n, jnp.int32)])
    gidx = gidx.reshape(_NW, nch, _CH)
    didx = didx.reshape(_NW, nch, _CH)

    rows_per_tile = -(-(-(-n // _NS)) // 8) * 8
    acc_rows = _NS * rows_per_tile
    zeros = jnp.zeros((rows_per_tile, out_feat), jnp.float32)

    table = _build_table(feat, w_all, b_all, bn=1000).reshape(-1, out_feat)

    edge_kernel = _make_edge_kernel(out_feat, nch, acc_rows, rows_per_tile)
    partials = edge_kernel(table, gidx, didx, zeros)[:, :n, :]

    out = _combine(partials, bn=1000)
    return out[:, None, :]


# speedup vs baseline: 9.2123x; 1.0143x over previous
import functools

import jax
import jax.numpy as jnp
from jax import lax
from jax.experimental import pallas as pl
from jax.experimental.pallas import tpu as pltpu
from jax.experimental.pallas import tpu_sc as plsc

_NC = 2
_NS = 16
_NW = _NC * _NS
_CH = 128


def _project_kernel(f_ref, w_ref, b_ref, y_ref):
    y = jnp.dot(f_ref[...], w_ref[...],
                preferred_element_type=jnp.float32,
                precision=lax.Precision.HIGHEST)
    y_ref[...] = y + b_ref[...]


def _build_table(feat, w_cat, b_cat, bn):
    n, in_feat = feat.shape
    kout = w_cat.shape[1]
    return pl.pallas_call(
        _project_kernel,
        grid=(n // bn,),
        in_specs=[
            pl.BlockSpec((bn, in_feat), lambda i: (i, 0)),
            pl.BlockSpec((in_feat, kout), lambda i: (0, 0)),
            pl.BlockSpec((1, kout), lambda i: (0, 0)),
        ],
        out_specs=pl.BlockSpec((bn, kout), lambda i: (i, 0)),
        out_shape=jax.ShapeDtypeStruct((n, kout), jnp.float32),
    )(feat, w_cat, b_cat)


def _combine_kernel(p_ref, o_ref):
    o_ref[...] = p_ref[0] + p_ref[1]


def _combine(partials, bn):
    _, n, out_feat = partials.shape
    return pl.pallas_call(
        _combine_kernel,
        grid=(n // bn,),
        in_specs=[pl.BlockSpec((2, bn, out_feat), lambda i: (0, i, 0))],
        out_specs=pl.BlockSpec((bn, out_feat), lambda i: (i, 0)),
        out_shape=jax.ShapeDtypeStruct((n, out_feat), jnp.float32),
    )(partials)


def _make_edge_kernel(out_feat, nch, acc_rows, rows_per_tile):
    mesh = plsc.VectorSubcoreMesh(core_axis_name="c", subcore_axis_name="s")

    @functools.partial(
        pl.kernel,
        mesh=mesh,
        out_type=jax.ShapeDtypeStruct((_NC, acc_rows, out_feat), jnp.float32),
        scratch_types=[
            pltpu.VMEM((nch, _CH), jnp.int32),
            pltpu.VMEM((nch, _CH), jnp.int32),
            pltpu.VMEM((_CH, out_feat), jnp.float32),
            pltpu.VMEM_SHARED((acc_rows, out_feat), jnp.float32),
            pltpu.SemaphoreType.DMA,
        ],
    )
    def edge_kernel(table_hbm, gidx_hbm, didx_hbm, zeros_hbm, out_hbm,
                    gidx_v, didx_v, rows_v, acc, sem):
        c = lax.axis_index("c")
        s = lax.axis_index("s")
        wid = s * _NC + c

        z0 = s * rows_per_tile
        pltpu.sync_copy(zeros_hbm.at[pl.ds(0, rows_per_tile)],
                        acc.at[pl.ds(z0, rows_per_tile)])
        pltpu.sync_copy(gidx_hbm.at[wid], gidx_v)
        pltpu.sync_copy(didx_hbm.at[wid], didx_v)
        plsc.subcore_barrier()

        def body(j, carry):
            pltpu.async_copy(table_hbm.at[gidx_v.at[j]], rows_v, sem).wait()
            pltpu.sync_copy(rows_v, acc.at[didx_v.at[j]], add=True)
            return carry

        lax.fori_loop(0, nch, body, 0)
        plsc.subcore_barrier()

        r0 = s * rows_per_tile
        pltpu.sync_copy(acc.at[pl.ds(r0, rows_per_tile)],
                        out_hbm.at[c, pl.ds(r0, rows_per_tile)])

    return edge_kernel


def kernel(feat, edge_index, etypes, ntypes, weight, m_bias, loop_weight, h_bias):
    n, in_feat = feat.shape
    num_rels, _, out_feat = weight.shape
    num_nt = loop_weight.shape[0]
    e = edge_index.shape[1]

    nk = num_rels + num_nt
    w_cat = jnp.concatenate([weight, loop_weight], axis=0)
    w_cat = w_cat.transpose(1, 0, 2).reshape(in_feat, nk * out_feat)
    b_cat = jnp.concatenate([m_bias, h_bias[:, 0, :]], axis=0).reshape(1, -1)

    src = edge_index[0].astype(jnp.int32)
    dst = edge_index[1].astype(jnp.int32)
    node_ids = jnp.arange(n, dtype=jnp.int32)
    gidx = jnp.concatenate([
        src * nk + etypes.astype(jnp.int32),
        node_ids * nk + (num_rels + ntypes.astype(jnp.int32)),
    ])
    didx = jnp.concatenate([dst, node_ids])

    rows_per_tile = -(-(-(-n // _NS)) // 8) * 8
    acc_rows = _NS * rows_per_tile
    zeros = jnp.zeros((rows_per_tile, out_feat), jnp.float32)

    e_tot = e + n
    pw = -(-e_tot // (_NW * _CH)) * _CH
    nch = pw // _CH
    ep = _NW * pw
    pad = ep - e_tot
    gidx = jnp.concatenate([gidx, jnp.zeros((pad,), jnp.int32)])
    didx = jnp.concatenate(
        [didx, n + jnp.arange(pad, dtype=jnp.int32) % (acc_rows - n)])
    gidx = gidx.reshape(_NW, nch, _CH)
    didx = didx.reshape(_NW, nch, _CH)

    table = _build_table(feat, w_cat, b_cat, bn=2000).reshape(-1, out_feat)

    edge_kernel = _make_edge_kernel(out_feat, nch, acc_rows, rows_per_tile)
    partials = edge_kernel(table, gidx, didx, zeros)[:, :n, :]

    out = _combine(partials, bn=1000)
    return out[:, None, :]
